# baseline (speedup 1.0000x reference)
import functools

import jax
import jax.numpy as jnp
from jax import lax
from jax.experimental import pallas as pl
from jax.experimental.pallas import tpu as pltpu
from jax.experimental.pallas import tpu_sc as plsc

B = 16384
EMB = 128
NW = 32
CHUNK = 128
SPLIT = 2
BS = B // SPLIT
NCHUNK = BS // NW // CHUNK


def _sc_gather(uids2, iids2, Ug, Ig, Um, Im):
    mesh = plsc.VectorSubcoreMesh(core_axis_name="c", subcore_axis_name="s")
    bpw = NCHUNK * CHUNK

    def body(u_hbm, i_hbm, ug_hbm, ig_hbm, um_hbm, im_hbm,
             out_ug, out_ig, out_um, out_im,
             uidx, iidx, buf0, buf1, sem0, sem1):
        wid = lax.axis_index("s") * 2 + lax.axis_index("c")
        idx_row0 = wid * NCHUNK
        pltpu.sync_copy(u_hbm.at[pl.ds(idx_row0, NCHUNK)], uidx)
        pltpu.sync_copy(i_hbm.at[pl.ds(idx_row0, NCHUNK)], iidx)

        jobs = []
        for tbl, idx, out in ((ug_hbm, uidx, out_ug), (ig_hbm, iidx, out_ig),
                              (um_hbm, uidx, out_um), (im_hbm, iidx, out_im)):
            for c in range(NCHUNK):
                jobs.append((tbl, idx, c, out))
        bufs = (buf0, buf1)
        sems = (sem0, sem1)
        row0 = wid * bpw
        pending = None
        for j, (tbl, idx, c, out) in enumerate(jobs):
            cp = pltpu.async_copy(tbl.at[idx.at[c]], bufs[j % 2], sems[j % 2])
            if pending is not None:
                pj, pcp, pout, pc = pending
                pcp.wait()
                pltpu.sync_copy(bufs[pj % 2],
                                pout.at[pl.ds(row0 + pc * CHUNK, CHUNK)])
            pending = (j, cp, out, c)
        pj, pcp, pout, pc = pending
        pcp.wait()
        pltpu.sync_copy(bufs[pj % 2], pout.at[pl.ds(row0 + pc * CHUNK, CHUNK)])

    return pl.kernel(
        body,
        out_type=[jax.ShapeDtypeStruct((BS, EMB), jnp.float32)] * 4,
        mesh=mesh,
        scratch_types=[
            pltpu.VMEM((NCHUNK, CHUNK), jnp.int32),
            pltpu.VMEM((NCHUNK, CHUNK), jnp.int32),
            pltpu.VMEM((CHUNK, EMB), jnp.float32),
            pltpu.VMEM((CHUNK, EMB), jnp.float32),
            pltpu.SemaphoreType.DMA,
            pltpu.SemaphoreType.DMA,
        ],
    )(uids2, iids2, Ug, Ig, Um, Im)


def _mlp_body(ug_r, ig_r, um_r, im_r, w1a_r, w1b_r, b1_r, w2_r, b2_r,
              w3_r, b3_r, wfa_r, wfb_r, bf_r, out_r):
    f32 = jnp.float32
    h1 = (jnp.dot(um_r[...], w1a_r[...], preferred_element_type=f32)
          + jnp.dot(im_r[...], w1b_r[...], preferred_element_type=f32)
          + b1_r[...])
    h1 = jnp.maximum(h1, 0.0)
    h2 = jnp.maximum(
        jnp.dot(h1, w2_r[...], preferred_element_type=f32) + b2_r[...], 0.0)
    h3 = jnp.maximum(
        jnp.dot(h2, w3_r[...], preferred_element_type=f32) + b3_r[...], 0.0)
    z = (jnp.sum(ug_r[...] * ig_r[...] * wfa_r[...], axis=1)
         + jnp.sum(h3 * wfb_r[...], axis=1) + bf_r[0, 0])
    out_r[...] = 1.0 / (1.0 + jnp.exp(-z))


def _tc_mlp(ug, ig, um, im, W1a, W1b, b1, W2, b2, W3, b3, wfa_row, wfb_row,
            bf):
    R = 512
    grid = (BS // R,)
    row_spec = pl.BlockSpec((R, EMB), lambda i: (i, 0))

    def fixed(shape):
        return pl.BlockSpec(shape, lambda i: tuple(0 for _ in shape))

    return pl.pallas_call(
        _mlp_body,
        grid=grid,
        in_specs=[
            row_spec, row_spec, row_spec, row_spec,
            fixed((EMB, 256)), fixed((EMB, 256)), fixed((1, 256)),
            fixed((256, 128)), fixed((1, 128)),
            fixed((128, 64)), fixed((1, 64)),
            fixed((1, EMB)), fixed((1, 64)), fixed((1, 1)),
        ],
        out_specs=pl.BlockSpec((R,), lambda i: (i,)),
        out_shape=jax.ShapeDtypeStruct((BS,), jnp.float32),
    )(ug, ig, um, im, W1a, W1b, b1, W2, b2, W3, b3, wfa_row, wfb_row, bf)


def kernel(user_ids, item_ids, Ug, Ig, Um, Im, W1, b1, W2, b2, W3, b3, Wf, bf):
    uids2 = user_ids.astype(jnp.int32).reshape(B // CHUNK, CHUNK)
    iids2 = item_ids.astype(jnp.int32).reshape(B // CHUNK, CHUNK)
    W1a, W1b = W1[:EMB], W1[EMB:]
    b1r, b2r, b3r = b1.reshape(1, -1), b2.reshape(1, -1), b3.reshape(1, -1)
    wfa_row = Wf[:EMB].reshape(1, EMB)
    wfb_row = Wf[EMB:].reshape(1, 64)
    bfr = bf.reshape(1, 1)

    rows_per_slice = BS // CHUNK
    outs = []
    for s in range(SPLIT):
        u_s = uids2[s * rows_per_slice:(s + 1) * rows_per_slice]
        i_s = iids2[s * rows_per_slice:(s + 1) * rows_per_slice]
        ug, ig, um, im = _sc_gather(u_s, i_s, Ug, Ig, Um, Im)
        outs.append(_tc_mlp(ug, ig, um, im, W1a, W1b, b1r, W2, b2r, W3, b3r,
                            wfa_row, wfb_row, bfr))
    return jnp.concatenate(outs).reshape(B, 1)


# speedup vs baseline: 1.1249x; 1.1249x over previous
import functools

import jax
import jax.numpy as jnp
from jax import lax
from jax.experimental import pallas as pl
from jax.experimental.pallas import tpu as pltpu
from jax.experimental.pallas import tpu_sc as plsc

B = 16384
EMB = 128
NW = 32
CHUNK = 128
SPLIT = 2
BS = B // SPLIT
NCHUNK = BS // NW // CHUNK


def _sc_gather(slice_idx, uids2, iids2, Ug, Ig, Um, Im):
    mesh = plsc.VectorSubcoreMesh(core_axis_name="c", subcore_axis_name="s")
    bpw = NCHUNK * CHUNK
    slice_row0 = slice_idx * (BS // CHUNK)

    def body(u_hbm, i_hbm, ug_hbm, ig_hbm, um_hbm, im_hbm,
             out_ug, out_ig, out_um, out_im,
             uidx, iidx, buf0, buf1, sem0, sem1):
        wid = lax.axis_index("s") * 2 + lax.axis_index("c")
        idx_row0 = slice_row0 + wid * NCHUNK
        pltpu.sync_copy(u_hbm.at[pl.ds(idx_row0, NCHUNK)], uidx)
        pltpu.sync_copy(i_hbm.at[pl.ds(idx_row0, NCHUNK)], iidx)

        jobs = []
        for tbl, idx, out in ((ug_hbm, uidx, out_ug), (ig_hbm, iidx, out_ig),
                              (um_hbm, uidx, out_um), (im_hbm, iidx, out_im)):
            for c in range(NCHUNK):
                jobs.append((tbl, idx, c, out))
        bufs = (buf0, buf1)
        sems = (sem0, sem1)
        row0 = wid * bpw
        pending = None
        for j, (tbl, idx, c, out) in enumerate(jobs):
            cp = pltpu.async_copy(tbl.at[idx.at[c]], bufs[j % 2], sems[j % 2])
            if pending is not None:
                pj, pcp, pout, pc = pending
                pcp.wait()
                pltpu.sync_copy(bufs[pj % 2],
                                pout.at[pl.ds(row0 + pc * CHUNK, CHUNK)])
            pending = (j, cp, out, c)
        pj, pcp, pout, pc = pending
        pcp.wait()
        pltpu.sync_copy(bufs[pj % 2], pout.at[pl.ds(row0 + pc * CHUNK, CHUNK)])

    return pl.kernel(
        body,
        out_type=[jax.ShapeDtypeStruct((BS, EMB), jnp.float32)] * 4,
        mesh=mesh,
        scratch_types=[
            pltpu.VMEM((NCHUNK, CHUNK), jnp.int32),
            pltpu.VMEM((NCHUNK, CHUNK), jnp.int32),
            pltpu.VMEM((CHUNK, EMB), jnp.float32),
            pltpu.VMEM((CHUNK, EMB), jnp.float32),
            pltpu.SemaphoreType.DMA,
            pltpu.SemaphoreType.DMA,
        ],
    )(uids2, iids2, Ug, Ig, Um, Im)


def _mlp_body(prev_r, ug_r, ig_r, um_r, im_r, w1a_r, w1b_r, b1_r, w2_r, b2_r,
              w3_r, b3_r, wfa_r, wfb_r, bf_r, out_r):
    del prev_r
    f32 = jnp.float32
    h1 = (jnp.dot(um_r[...], w1a_r[...], preferred_element_type=f32)
          + jnp.dot(im_r[...], w1b_r[...], preferred_element_type=f32)
          + b1_r[...])
    h1 = jnp.maximum(h1, 0.0)
    h2 = jnp.maximum(
        jnp.dot(h1, w2_r[...], preferred_element_type=f32) + b2_r[...], 0.0)
    h3 = jnp.maximum(
        jnp.dot(h2, w3_r[...], preferred_element_type=f32) + b3_r[...], 0.0)
    z = (jnp.sum(ug_r[...] * ig_r[...] * wfa_r[...], axis=1)
         + jnp.sum(h3 * wfb_r[...], axis=1) + bf_r[0, 0])
    out_r[...] = 1.0 / (1.0 + jnp.exp(-z))


def _tc_mlp(slice_idx, prev, ug, ig, um, im, W1a, W1b, b1, W2, b2, W3, b3,
            wfa_row, wfb_row, bf):
    R = 1024
    grid = (BS // R,)
    base = slice_idx * (BS // R)
    row_spec = pl.BlockSpec((R, EMB), lambda i: (i, 0))
    out_spec = pl.BlockSpec((R,), lambda i: (base + i,))

    def fixed(shape):
        return pl.BlockSpec(shape, lambda i: tuple(0 for _ in shape))

    return pl.pallas_call(
        _mlp_body,
        grid=grid,
        in_specs=[
            out_spec,
            row_spec, row_spec, row_spec, row_spec,
            fixed((EMB, 256)), fixed((EMB, 256)), fixed((1, 256)),
            fixed((256, 128)), fixed((1, 128)),
            fixed((128, 64)), fixed((1, 64)),
            fixed((1, EMB)), fixed((1, 64)), fixed((1, 1)),
        ],
        out_specs=out_spec,
        out_shape=jax.ShapeDtypeStruct((B,), jnp.float32),
        input_output_aliases={0: 0},
    )(prev, ug, ig, um, im, W1a, W1b, b1, W2, b2, W3, b3, wfa_row, wfb_row,
      bf)


def kernel(user_ids, item_ids, Ug, Ig, Um, Im, W1, b1, W2, b2, W3, b3, Wf, bf):
    uids2 = user_ids.astype(jnp.int32).reshape(B // CHUNK, CHUNK)
    iids2 = item_ids.astype(jnp.int32).reshape(B // CHUNK, CHUNK)
    W1a, W1b = W1[:EMB], W1[EMB:]
    b1r, b2r, b3r = b1.reshape(1, -1), b2.reshape(1, -1), b3.reshape(1, -1)
    wfa_row = Wf[:EMB].reshape(1, EMB)
    wfb_row = Wf[EMB:].reshape(1, 64)
    bfr = bf.reshape(1, 1)

    out = jnp.zeros((B,), jnp.float32)
    for s in range(SPLIT):
        ug, ig, um, im = _sc_gather(s, uids2, iids2, Ug, Ig, Um, Im)
        out = _tc_mlp(s, out, ug, ig, um, im, W1a, W1b, b1r, W2, b2r, W3,
                      b3r, wfa_row, wfb_row, bfr)
    return out.reshape(B, 1)


# speedup vs baseline: 1.1250x; 1.0001x over previous
import jax
import jax.numpy as jnp
from jax import lax
from jax.experimental import pallas as pl
from jax.experimental.pallas import tpu as pltpu
from jax.experimental.pallas import tpu_sc as plsc

B = 16384
EMB = 128
NW = 32
CHUNK = 128
SPLIT = 2
BS = B // SPLIT
NCHUNK = BS // NW // CHUNK


def _sc_gather(slice_idx, uids2, iids2, Ug, Ig, Um, Im):
    mesh = plsc.VectorSubcoreMesh(core_axis_name="c", subcore_axis_name="s")
    bpw = NCHUNK * CHUNK
    slice_row0 = slice_idx * (BS // CHUNK)

    def body(u_hbm, i_hbm, ug_hbm, ig_hbm, um_hbm, im_hbm,
             out_ug, out_ig, out_um, out_im,
             uidx, iidx, buf0, buf1, sem0, sem1):
        wid = lax.axis_index("s") * 2 + lax.axis_index("c")
        idx_row0 = slice_row0 + wid * NCHUNK
        pltpu.sync_copy(u_hbm.at[pl.ds(idx_row0, NCHUNK)], uidx)
        pltpu.sync_copy(i_hbm.at[pl.ds(idx_row0, NCHUNK)], iidx)

        jobs = []
        for tbl, idx, out in ((ug_hbm, uidx, out_ug), (ig_hbm, iidx, out_ig),
                              (um_hbm, uidx, out_um), (im_hbm, iidx, out_im)):
            for c in range(NCHUNK):
                jobs.append((tbl, idx, c, out))
        bufs = (buf0, buf1)
        sems = (sem0, sem1)
        row0 = wid * bpw
        pending = None
        for j, (tbl, idx, c, out) in enumerate(jobs):
            cp = pltpu.async_copy(tbl.at[idx.at[c]], bufs[j % 2], sems[j % 2])
            if pending is not None:
                pj, pcp, pout, pc = pending
                pcp.wait()
                pltpu.sync_copy(bufs[pj % 2],
                                pout.at[pl.ds(row0 + pc * CHUNK, CHUNK)])
            pending = (j, cp, out, c)
        pj, pcp, pout, pc = pending
        pcp.wait()
        pltpu.sync_copy(bufs[pj % 2], pout.at[pl.ds(row0 + pc * CHUNK, CHUNK)])

    return pl.kernel(
        body,
        out_type=[jax.ShapeDtypeStruct((BS, EMB), jnp.float32)] * 4,
        mesh=mesh,
        scratch_types=[
            pltpu.VMEM((NCHUNK, CHUNK), jnp.int32),
            pltpu.VMEM((NCHUNK, CHUNK), jnp.int32),
            pltpu.VMEM((CHUNK, EMB), jnp.float32),
            pltpu.VMEM((CHUNK, EMB), jnp.float32),
            pltpu.SemaphoreType.DMA,
            pltpu.SemaphoreType.DMA,
        ],
    )(uids2, iids2, Ug, Ig, Um, Im)


def _mlp_body(prev_r, ug_r, ig_r, um_r, im_r, w1a_r, w1b_r, b1_r, w2_r, b2_r,
              w3_r, b3_r, wfa_r, wfb_r, bf_r, out_r):
    del prev_r
    f32 = jnp.float32
    h1 = (jnp.dot(um_r[...], w1a_r[...], preferred_element_type=f32)
          + jnp.dot(im_r[...], w1b_r[...], preferred_element_type=f32)
          + b1_r[...])
    h1 = jnp.maximum(h1, 0.0)
    h2 = jnp.maximum(
        jnp.dot(h1, w2_r[...], preferred_element_type=f32) + b2_r[...], 0.0)
    h3 = jnp.maximum(
        jnp.dot(h2, w3_r[...], preferred_element_type=f32) + b3_r[...], 0.0)
    z = (jnp.sum(ug_r[...] * ig_r[...] * wfa_r[...], axis=1)
         + jnp.sum(h3 * wfb_r[...], axis=1) + bf_r[0, 0])
    out_r[...] = 1.0 / (1.0 + jnp.exp(-z))


def _tc_mlp(slice_idx, prev, ug, ig, um, im, W1a, W1b, b1, W2, b2, W3, b3,
            wfa_row, wfb_row, bf):
    R = 1024
    grid = (BS // R,)
    base = slice_idx * (BS // R)
    row_spec = pl.BlockSpec((R, EMB), lambda i: (i, 0))
    out_spec = pl.BlockSpec((R,), lambda i: (base + i,))

    def fixed(shape):
        return pl.BlockSpec(shape, lambda i: tuple(0 for _ in shape))

    return pl.pallas_call(
        _mlp_body,
        grid=grid,
        in_specs=[
            out_spec,
            row_spec, row_spec, row_spec, row_spec,
            fixed((EMB, 256)), fixed((EMB, 256)), fixed((1, 256)),
            fixed((256, 128)), fixed((1, 128)),
            fixed((128, 64)), fixed((1, 64)),
            fixed((1, EMB)), fixed((1, 64)), fixed((1, 1)),
        ],
        out_specs=out_spec,
        out_shape=jax.ShapeDtypeStruct((B,), jnp.float32),
        input_output_aliases={0: 0},
    )(prev, ug, ig, um, im, W1a, W1b, b1, W2, b2, W3, b3, wfa_row, wfb_row,
      bf)


def kernel(user_ids, item_ids, Ug, Ig, Um, Im, W1, b1, W2, b2, W3, b3, Wf, bf):
    uids2 = user_ids.astype(jnp.int32).reshape(B // CHUNK, CHUNK)
    iids2 = item_ids.astype(jnp.int32).reshape(B // CHUNK, CHUNK)
    W1a, W1b = W1[:EMB], W1[EMB:]
    b1r, b2r, b3r = b1.reshape(1, -1), b2.reshape(1, -1), b3.reshape(1, -1)
    wfa_row = Wf[:EMB].reshape(1, EMB)
    wfb_row = Wf[EMB:].reshape(1, 64)
    bfr = bf.reshape(1, 1)

    out = jnp.zeros((B,), jnp.float32)
    for s in range(SPLIT):
        ug, ig, um, im = _sc_gather(s, uids2, iids2, Ug, Ig, Um, Im)
        out = _tc_mlp(s, out, ug, ig, um, im, W1a, W1b, b1r, W2, b2r, W3,
                      b3r, wfa_row, wfb_row, bfr)
    return out.reshape(B, 1)
